# initial kernel scaffold (unmeasured)
import jax
import jax.numpy as jnp
from jax import lax
from jax.experimental import pallas as pl
from jax.experimental.pallas import tpu as pltpu


def kernel(
    x,
):
    def body(*refs):
        pass

    out_shape = jax.ShapeDtypeStruct(..., jnp.float32)
    return pl.pallas_call(body, out_shape=out_shape)(...)


# baseline (device time: 6897898 ns/iter reference)
import jax
import jax.numpy as jnp
from jax import lax
from jax.experimental import pallas as pl
from jax.experimental.pallas import tpu as pltpu

N_DEV = 4
R = 8192
C = 1024
BC = 128
NB = C // BC
N_SORT_PASSES = 91


def _ce(o_ref, jj, asc_m, iota):
    v = o_ref[...]
    up = pltpu.roll(v, jj, axis=0)
    dn = pltpu.roll(v, R - jj, axis=0)
    bit = (iota & jj) != 0
    p = jnp.where(bit, up, dn)
    take_lo = jnp.logical_xor(asc_m, bit)
    o_ref[...] = jnp.where(take_lo, jnp.minimum(v, p), jnp.maximum(v, p))


def kernel(x):
    def body(x_ref, o_ref, comm_ref, send_sems, recv_sems):
        d = lax.axis_index("i")
        p1 = d ^ 1
        p2 = d ^ 2

        barrier = pltpu.get_barrier_semaphore()
        for nbr in (p1, p2):
            pl.semaphore_signal(
                barrier, inc=1,
                device_id=(nbr,), device_id_type=pl.DeviceIdType.MESH,
            )
        pl.semaphore_wait(barrier, 2)

        iota = lax.broadcasted_iota(jnp.int32, (R, 1), 0)

        asc_dev = (d & 1) == 0
        o_ref[...] = x_ref[...].astype(jnp.bfloat16)

        def sort_pass(_, kj):
            kk, jj = kj
            asc_m = jnp.logical_xor((iota & kk) != 0, asc_dev)
            _ce(o_ref, jj, asc_m, iota)
            last = jj == 1
            return (
                jnp.where(last, kk << 1, kk),
                jnp.where(last, kk, jj >> 1),
            )

        lax.fori_loop(
            0, N_SORT_PASSES, sort_pass,
            (jnp.int32(2), jnp.int32(1)),
        )

        def merge_local(asc_scalar):
            def merge_pass(_, jj):
                _ce(o_ref, jj, asc_scalar, iota)
                return jj >> 1
            lax.fori_loop(0, 13, merge_pass, jnp.int32(R // 2))

        def exchange(slot, partner, keep_min):
            rdma = pltpu.make_async_remote_copy(
                src_ref=o_ref,
                dst_ref=comm_ref.at[slot],
                send_sem=send_sems.at[slot],
                recv_sem=recv_sems.at[slot],
                device_id=(partner,),
                device_id_type=pl.DeviceIdType.MESH,
            )
            rdma.start()
            rdma.wait()
            v = o_ref[...]
            c = comm_ref[slot]
            o_ref[...] = jnp.where(
                keep_min, jnp.minimum(v, c), jnp.maximum(v, c)
            )

        asc2 = (d & 2) == 0
        keep1 = jnp.logical_xor(asc2, (d & 1) != 0)
        exchange(0, p1, keep1)
        merge_local(asc2)

        exchange(1, p2, (d & 2) == 0)
        exchange(2, p1, (d & 1) == 0)
        merge_local(jnp.bool_(True))

    return pl.pallas_call(
        body,
        grid=(NB,),
        in_specs=[pl.BlockSpec((R, BC), lambda g: (0, g))],
        out_specs=pl.BlockSpec((R, BC), lambda g: (0, g)),
        out_shape=jax.ShapeDtypeStruct((R, C), jnp.bfloat16),
        scratch_shapes=[
            pltpu.VMEM((3, R, BC), jnp.bfloat16),
            pltpu.SemaphoreType.DMA((3,)),
            pltpu.SemaphoreType.DMA((3,)),
        ],
        compiler_params=pltpu.CompilerParams(
            collective_id=0,
            dimension_semantics=("arbitrary",),
            vmem_limit_bytes=64 * 1024 * 1024,
        ),
    )(x)
